# baseline (speedup 1.0000x reference)
import functools

import jax
import jax.numpy as jnp
from jax import lax
from jax.experimental import pallas as pl
from jax.experimental.pallas import tpu as pltpu
from jax.experimental.pallas import tpu_sc as plsc

N = 10000
E = 320000
D_FEAT = 128
D_HID = 16

NPAD = 10240
NW = 32
SBLK = 1000
NSB = 10
EPW = NSB * SBLK
STRIPE = NPAD // 16

_mesh = plsc.VectorSubcoreMesh(core_axis_name="c", subcore_axis_name="s")
_sc_params = pltpu.CompilerParams(use_tc_tiling_on_sc=False)
_f32 = jnp.float32


def _zero_shared(z_hbm, shared, s):
    pltpu.sync_copy(z_hbm.at[pl.ds(s * STRIPE, STRIPE)],
                    shared.at[pl.ds(s * STRIPE, STRIPE)])


def _flush_shared(shared, out_hbm, c, s):
    pltpu.sync_copy(shared.at[pl.ds(s * STRIPE, STRIPE)],
                    out_hbm.at[c, pl.ds(s * STRIPE, STRIPE)])


def _rsqrt_newton(d):
    i = lax.bitcast_convert_type(d, jnp.int32)
    y = lax.bitcast_convert_type(0x5F3759DF - (i >> 1), _f32)
    for _ in range(3):
        y = y * (1.5 - 0.5 * d * y * y)
    return y


def _edge_phase(src_v, dst_v, bufs, gsems, ssems, shared, shared_g):
    def gather(i):
        return pltpu.async_copy(shared_g.at[src_v.at[i]], bufs[i % 4],
                                gsems[i % 4])

    def scatter(i):
        return pltpu.async_copy(bufs[i % 4], shared.at[dst_v.at[i]],
                                ssems[i % 4], add=True)

    for i in range(3):
        gather(i)
    for i in range(NSB):
        if i + 3 < NSB:
            if i >= 1:
                pltpu.make_async_copy(bufs[(i - 1) % 4],
                                      shared.at[dst_v.at[i - 1]],
                                      ssems[(i - 1) % 4]).wait()
            gather(i + 3)
        pltpu.make_async_copy(shared_g.at[src_v.at[i]], bufs[i % 4],
                              gsems[i % 4]).wait()
        scatter(i)
    for i in range(NSB - 4, NSB):
        pltpu.make_async_copy(bufs[i % 4], shared.at[dst_v.at[i]],
                              ssems[i % 4]).wait()


@functools.partial(
    pl.kernel,
    out_type=jax.ShapeDtypeStruct((2, NPAD, D_HID), _f32),
    mesh=_mesh,
    scratch_types=[
        pltpu.VMEM((NSB, SBLK), jnp.int32),
        pltpu.VMEM((SBLK, D_HID), _f32),
        pltpu.VMEM_SHARED((NPAD, D_HID), _f32),
        pltpu.SemaphoreType.DMA,
    ],
    compiler_params=_sc_params,
)
def _deg_pass(dst_hbm, z_hbm, ones_hbm, out_hbm, dst_v, ones_v, shared, sem):
    c = lax.axis_index("c")
    s = lax.axis_index("s")
    w = c * 16 + s
    pltpu.sync_copy(dst_hbm.at[w], dst_v)
    pltpu.sync_copy(ones_hbm, ones_v)
    _zero_shared(z_hbm, shared, s)
    plsc.subcore_barrier()

    for i in range(NSB):
        pltpu.async_copy(ones_v, shared.at[dst_v.at[i]], sem, add=True)
    for i in range(NSB):
        pltpu.make_async_copy(ones_v, shared.at[dst_v.at[i]], sem).wait()
    plsc.subcore_barrier()
    _flush_shared(shared, out_hbm, c, s)


@functools.partial(
    pl.kernel,
    out_type=(jax.ShapeDtypeStruct((2, NPAD, D_HID), _f32),
              jax.ShapeDtypeStruct((NPAD, D_HID), _f32),
              jax.ShapeDtypeStruct((NPAD, D_HID), _f32)),
    mesh=_mesh,
    scratch_types=[
        pltpu.VMEM((NSB, SBLK), jnp.int32),
        pltpu.VMEM((NSB, SBLK), jnp.int32),
        [pltpu.VMEM((SBLK, D_HID), _f32) for _ in range(4)],
        [pltpu.SemaphoreType.DMA for _ in range(4)],
        [pltpu.SemaphoreType.DMA for _ in range(4)],
        pltpu.VMEM_SHARED((NPAD, D_HID), _f32),
        pltpu.VMEM_SHARED((NPAD, D_HID), _f32),
    ],
    compiler_params=_sc_params,
)
def _edge1(h1_hbm, degp_hbm, src_hbm, dst_hbm, z_hbm,
           sp_hbm, g1_hbm, dinv_hbm,
           src_v, dst_v, bufs, gsems, ssems, shared, shared_g):
    c = lax.axis_index("c")
    s = lax.axis_index("s")
    w = c * 16 + s
    pltpu.sync_copy(src_hbm.at[w], src_v)
    pltpu.sync_copy(dst_hbm.at[w], dst_v)

    st = pl.ds(s * STRIPE, STRIPE)
    half = pl.ds(0, STRIPE)
    pltpu.sync_copy(degp_hbm.at[0, st], bufs[0].at[half])
    pltpu.sync_copy(degp_hbm.at[1, st], bufs[1].at[half])
    pltpu.sync_copy(h1_hbm.at[st], bufs[2].at[half])

    def row(r, carry):
        d = bufs[0][r] + bufs[1][r] + 1.0
        dinv = _rsqrt_newton(d)
        bufs[0][r] = dinv
        bufs[3][r] = dinv * bufs[2][r]
        return carry

    lax.fori_loop(0, STRIPE, row, 0, unroll=8)
    pltpu.sync_copy(bufs[3].at[half], shared_g.at[st])
    pltpu.sync_copy(bufs[3].at[half], g1_hbm.at[st])
    pltpu.sync_copy(bufs[0].at[half], dinv_hbm.at[st])
    _zero_shared(z_hbm, shared, s)
    plsc.subcore_barrier()

    _edge_phase(src_v, dst_v, bufs, gsems, ssems, shared, shared_g)
    plsc.subcore_barrier()
    _flush_shared(shared, sp_hbm, c, s)


@functools.partial(
    pl.kernel,
    out_type=(jax.ShapeDtypeStruct((2, NPAD, D_HID), _f32),
              jax.ShapeDtypeStruct((NPAD, D_HID), _f32)),
    mesh=_mesh,
    scratch_types=[
        pltpu.VMEM((NSB, SBLK), jnp.int32),
        pltpu.VMEM((NSB, SBLK), jnp.int32),
        [pltpu.VMEM((SBLK, D_HID), _f32) for _ in range(4)],
        pltpu.VMEM((STRIPE, D_HID), _f32),
        pltpu.VMEM((D_HID, D_HID), _f32),
        pltpu.VMEM((1, D_HID), _f32),
        [pltpu.SemaphoreType.DMA for _ in range(4)],
        [pltpu.SemaphoreType.DMA for _ in range(4)],
        pltpu.VMEM_SHARED((NPAD, D_HID), _f32),
        pltpu.VMEM_SHARED((NPAD, D_HID), _f32),
    ],
    compiler_params=_sc_params,
)
def _edge2(sp1_hbm, g1_hbm, dinv_hbm, w2_hbm, b1_hbm, src_hbm, dst_hbm, z_hbm,
           sp_hbm, g2_hbm,
           src_v, dst_v, bufs, dinv_v, w2_v, b1_v, gsems, ssems,
           shared, shared_g):
    c = lax.axis_index("c")
    s = lax.axis_index("s")
    w = c * 16 + s
    pltpu.sync_copy(src_hbm.at[w], src_v)
    pltpu.sync_copy(dst_hbm.at[w], dst_v)

    st = pl.ds(s * STRIPE, STRIPE)
    half = pl.ds(0, STRIPE)
    pltpu.sync_copy(sp1_hbm.at[0, st], bufs[0].at[half])
    pltpu.sync_copy(sp1_hbm.at[1, st], bufs[1].at[half])
    pltpu.sync_copy(g1_hbm.at[st], bufs[2].at[half])
    pltpu.sync_copy(dinv_hbm.at[st], dinv_v)
    pltpu.sync_copy(w2_hbm, w2_v)
    pltpu.sync_copy(b1_hbm, b1_v)

    def zrow(r, carry):
        s1 = bufs[0][r] + bufs[1][r] + bufs[2][r]
        bufs[3][r] = jnp.maximum(dinv_v[r] * s1 + b1_v[0], 0.0)
        return carry

    lax.fori_loop(0, STRIPE, zrow, 0, unroll=8)

    def mmrow(r, carry):
        zv = bufs[3][r]
        accs = [zv[j] * w2_v[j] for j in range(4)]
        for k in range(4, D_HID):
            accs[k % 4] = accs[k % 4] + zv[k] * w2_v[k]
        acc = (accs[0] + accs[1]) + (accs[2] + accs[3])
        bufs[0][r] = acc * dinv_v[r]
        return carry

    lax.fori_loop(0, STRIPE, mmrow, 0, unroll=4)
    pltpu.sync_copy(bufs[0].at[half], shared_g.at[st])
    pltpu.sync_copy(bufs[0].at[half], g2_hbm.at[st])
    _zero_shared(z_hbm, shared, s)
    plsc.subcore_barrier()

    _edge_phase(src_v, dst_v, bufs, gsems, ssems, shared, shared_g)
    plsc.subcore_barrier()
    _flush_shared(shared, sp_hbm, c, s)


def _tc_matmul1(x_r, W1_bd):
    rows = NPAD // 8

    def body(x_ref, w_ref, h_ref):
        h_ref[...] = jnp.dot(x_ref[...], w_ref[...],
                             preferred_element_type=_f32)

    return pl.pallas_call(
        body,
        out_shape=jax.ShapeDtypeStruct((rows, 128), _f32),
    )(x_r, W1_bd)


def _tc_out(sp2, g2, dinv, b2):
    rows = NPAD * D_HID // 128
    lane = jnp.arange(128, dtype=jnp.int32)
    grp = lane // D_HID
    sel = (lane[:, None] == grp[None, :] * D_HID).astype(_f32)
    blk = (grp[:, None] == grp[None, :]).astype(_f32)

    def body(sp_ref, g_ref, dinv_ref, b_ref, sel_ref, blk_ref, out_ref):
        o = dinv_ref[...] * (sp_ref[0] + sp_ref[1] + g_ref[...]) + b_ref[...]
        m = o
        for k in (1, 2, 4, 8):
            m = jnp.maximum(m, pltpu.roll(m, 128 - k, 1))
        mb = jax.lax.dot(m, sel_ref[...], preferred_element_type=_f32)
        e = jnp.exp(o - mb)
        ssum = jax.lax.dot(e, blk_ref[...], preferred_element_type=_f32)
        out_ref[...] = (o - mb) - jnp.log(ssum)

    res = pl.pallas_call(
        body,
        out_shape=jax.ShapeDtypeStruct((rows, 128), _f32),
    )(sp2.reshape(2, rows, 128), g2.reshape(rows, 128),
      dinv.reshape(rows, 128), jnp.tile(b2, (1, 128 // D_HID)), sel, blk)
    return res.reshape(NPAD, D_HID)


def kernel(x, edge_index, W1, b1, W2, b2):
    dstp = edge_index[1].astype(jnp.int32).reshape(NW, NSB, SBLK)
    z_t = jnp.zeros((NPAD, D_HID), _f32)
    ones_t = jnp.ones((SBLK, D_HID), _f32)

    degp = _deg_pass(dstp, z_t, ones_t)
    srcp, dstp2, _ = lax.optimization_barrier(
        (edge_index[0].astype(jnp.int32).reshape(NW, NSB, SBLK), dstp, degp))

    x_r = jnp.pad(x, ((0, NPAD - N), (0, 0))).reshape(NPAD // 8, 8 * D_FEAT)
    W1_bd = jnp.kron(jnp.eye(8, dtype=_f32), W1)
    h1 = _tc_matmul1(x_r, W1_bd)
    sp1, g1, dinv = _edge1(h1.reshape(NPAD, D_HID), degp, srcp, dstp2, z_t)
    sp2, g2 = _edge2(sp1, g1, dinv, W2, b1.reshape(1, D_HID),
                     srcp, dstp2, z_t)
    out = _tc_out(sp2, g2, dinv, b2.reshape(1, D_HID))
    return out[:N]


# speedup vs baseline: 1.0725x; 1.0725x over previous
import functools

import jax
import jax.numpy as jnp
from jax import lax
from jax.experimental import pallas as pl
from jax.experimental.pallas import tpu as pltpu
from jax.experimental.pallas import tpu_sc as plsc

N = 10000
E = 320000
D_FEAT = 128
D_HID = 16

NPAD = 10240
NW = 32
SBLK = 1000
NSB = 10
EPW = NSB * SBLK
STRIPE = NPAD // 16

_mesh = plsc.VectorSubcoreMesh(core_axis_name="c", subcore_axis_name="s")
_sc_params = pltpu.CompilerParams(use_tc_tiling_on_sc=False)
_f32 = jnp.float32


def _fill(buf, n, val):
    v = jnp.full((D_HID,), val, _f32)

    def row(r, carry):
        buf[r] = v
        return carry

    lax.fori_loop(0, n, row, 0, unroll=8)


def _flush_shared(shared, out_hbm, c, s):
    pltpu.sync_copy(shared.at[pl.ds(s * STRIPE, STRIPE)],
                    out_hbm.at[c, pl.ds(s * STRIPE, STRIPE)])


def _rsqrt_newton(d):
    i = lax.bitcast_convert_type(d, jnp.int32)
    y = lax.bitcast_convert_type(0x5F3759DF - (i >> 1), _f32)
    for _ in range(3):
        y = y * (1.5 - 0.5 * d * y * y)
    return y


def _edge_phase(src_v, dst_v, bufs, gsems, ssems, shared, shared_g):
    def gather(i):
        return pltpu.async_copy(shared_g.at[src_v.at[i]], bufs[i % 4],
                                gsems[i % 4])

    def scatter(i):
        return pltpu.async_copy(bufs[i % 4], shared.at[dst_v.at[i]],
                                ssems[i % 4], add=True)

    for i in range(3):
        gather(i)
    for i in range(NSB):
        if i + 3 < NSB:
            if i >= 1:
                pltpu.make_async_copy(bufs[(i - 1) % 4],
                                      shared.at[dst_v.at[i - 1]],
                                      ssems[(i - 1) % 4]).wait()
            gather(i + 3)
        pltpu.make_async_copy(shared_g.at[src_v.at[i]], bufs[i % 4],
                              gsems[i % 4]).wait()
        scatter(i)
    for i in range(NSB - 4, NSB):
        pltpu.make_async_copy(bufs[i % 4], shared.at[dst_v.at[i]],
                              ssems[i % 4]).wait()


@functools.partial(
    pl.kernel,
    out_type=jax.ShapeDtypeStruct((2, NPAD, D_HID), _f32),
    mesh=_mesh,
    scratch_types=[
        pltpu.VMEM((NSB, SBLK), jnp.int32),
        pltpu.VMEM((SBLK, D_HID), _f32),
        pltpu.VMEM((STRIPE, D_HID), _f32),
        pltpu.VMEM_SHARED((NPAD, D_HID), _f32),
        pltpu.SemaphoreType.DMA,
        pltpu.SemaphoreType.DMA,
    ],
    compiler_params=_sc_params,
)
def _deg_pass(dst_hbm, out_hbm, dst_v, ones_v, zbuf, shared, sem, isem):
    c = lax.axis_index("c")
    s = lax.axis_index("s")
    w = c * 16 + s
    pltpu.async_copy(dst_hbm.at[w], dst_v, isem)
    _fill(ones_v, SBLK, 1.0)
    _fill(zbuf, STRIPE, 0.0)
    pltpu.sync_copy(zbuf, shared.at[pl.ds(s * STRIPE, STRIPE)])
    pltpu.make_async_copy(dst_hbm.at[w], dst_v, isem).wait()
    plsc.subcore_barrier()

    for i in range(NSB):
        pltpu.async_copy(ones_v, shared.at[dst_v.at[i]], sem, add=True)
    for i in range(NSB):
        pltpu.make_async_copy(ones_v, shared.at[dst_v.at[i]], sem).wait()
    plsc.subcore_barrier()
    _flush_shared(shared, out_hbm, c, s)


@functools.partial(
    pl.kernel,
    out_type=(jax.ShapeDtypeStruct((2, NPAD, D_HID), _f32),
              jax.ShapeDtypeStruct((NPAD, D_HID), _f32),
              jax.ShapeDtypeStruct((NPAD, D_HID), _f32)),
    mesh=_mesh,
    scratch_types=[
        pltpu.VMEM((NSB, SBLK), jnp.int32),
        pltpu.VMEM((NSB, SBLK), jnp.int32),
        [pltpu.VMEM((SBLK, D_HID), _f32) for _ in range(4)],
        [pltpu.SemaphoreType.DMA for _ in range(4)],
        [pltpu.SemaphoreType.DMA for _ in range(4)],
        [pltpu.SemaphoreType.DMA for _ in range(2)],
        pltpu.VMEM_SHARED((NPAD, D_HID), _f32),
        pltpu.VMEM_SHARED((NPAD, D_HID), _f32),
    ],
    compiler_params=_sc_params,
)
def _edge1(h1_hbm, degp_hbm, src_hbm, dst_hbm,
           sp_hbm, g1_hbm, dinv_hbm,
           src_v, dst_v, bufs, gsems, ssems, isems, shared, shared_g):
    c = lax.axis_index("c")
    s = lax.axis_index("s")
    w = c * 16 + s
    st = pl.ds(s * STRIPE, STRIPE)
    half = pl.ds(0, STRIPE)
    pltpu.async_copy(src_hbm.at[w], src_v, isems[0])
    pltpu.async_copy(dst_hbm.at[w], dst_v, isems[1])
    pltpu.async_copy(degp_hbm.at[0, st], bufs[0].at[half], gsems[0])
    pltpu.async_copy(degp_hbm.at[1, st], bufs[1].at[half], gsems[1])
    pltpu.async_copy(h1_hbm.at[st], bufs[2].at[half], gsems[2])
    _fill(bufs[3], STRIPE, 0.0)
    pltpu.sync_copy(bufs[3].at[half], shared.at[st])
    pltpu.make_async_copy(degp_hbm.at[0, st], bufs[0].at[half], gsems[0]).wait()
    pltpu.make_async_copy(degp_hbm.at[1, st], bufs[1].at[half], gsems[1]).wait()
    pltpu.make_async_copy(h1_hbm.at[st], bufs[2].at[half], gsems[2]).wait()

    def row(r, carry):
        d = bufs[0][r] + bufs[1][r] + 1.0
        dinv = _rsqrt_newton(d)
        bufs[0][r] = dinv
        bufs[3][r] = dinv * bufs[2][r]
        return carry

    lax.fori_loop(0, STRIPE, row, 0, unroll=8)
    pltpu.sync_copy(bufs[3].at[half], shared_g.at[st])
    pltpu.sync_copy(bufs[3].at[half], g1_hbm.at[st])
    pltpu.sync_copy(bufs[0].at[half], dinv_hbm.at[st])
    pltpu.make_async_copy(src_hbm.at[w], src_v, isems[0]).wait()
    pltpu.make_async_copy(dst_hbm.at[w], dst_v, isems[1]).wait()
    plsc.subcore_barrier()

    _edge_phase(src_v, dst_v, bufs, gsems, ssems, shared, shared_g)
    plsc.subcore_barrier()
    _flush_shared(shared, sp_hbm, c, s)


@functools.partial(
    pl.kernel,
    out_type=(jax.ShapeDtypeStruct((2, NPAD, D_HID), _f32),
              jax.ShapeDtypeStruct((NPAD, D_HID), _f32)),
    mesh=_mesh,
    scratch_types=[
        pltpu.VMEM((NSB, SBLK), jnp.int32),
        pltpu.VMEM((NSB, SBLK), jnp.int32),
        [pltpu.VMEM((SBLK, D_HID), _f32) for _ in range(4)],
        pltpu.VMEM((STRIPE, D_HID), _f32),
        pltpu.VMEM((D_HID, D_HID), _f32),
        pltpu.VMEM((1, D_HID), _f32),
        [pltpu.SemaphoreType.DMA for _ in range(4)],
        [pltpu.SemaphoreType.DMA for _ in range(4)],
        [pltpu.SemaphoreType.DMA for _ in range(4)],
        pltpu.VMEM_SHARED((NPAD, D_HID), _f32),
        pltpu.VMEM_SHARED((NPAD, D_HID), _f32),
    ],
    compiler_params=_sc_params,
)
def _edge2(sp1_hbm, g1_hbm, dinv_hbm, w2_hbm, b1_hbm, src_hbm, dst_hbm,
           sp_hbm, g2_hbm,
           src_v, dst_v, bufs, dinv_v, w2_v, b1_v, gsems, ssems, isems,
           shared, shared_g):
    c = lax.axis_index("c")
    s = lax.axis_index("s")
    w = c * 16 + s
    st = pl.ds(s * STRIPE, STRIPE)
    half = pl.ds(0, STRIPE)
    pltpu.async_copy(src_hbm.at[w], src_v, isems[0])
    pltpu.async_copy(dst_hbm.at[w], dst_v, isems[1])
    pltpu.async_copy(sp1_hbm.at[0, st], bufs[0].at[half], gsems[0])
    pltpu.async_copy(sp1_hbm.at[1, st], bufs[1].at[half], gsems[1])
    pltpu.async_copy(g1_hbm.at[st], bufs[2].at[half], gsems[2])
    pltpu.async_copy(dinv_hbm.at[st], dinv_v, gsems[3])
    pltpu.async_copy(w2_hbm, w2_v, isems[2])
    pltpu.async_copy(b1_hbm, b1_v, isems[3])
    _fill(bufs[3], STRIPE, 0.0)
    pltpu.sync_copy(bufs[3].at[half], shared.at[st])
    pltpu.make_async_copy(sp1_hbm.at[0, st], bufs[0].at[half], gsems[0]).wait()
    pltpu.make_async_copy(sp1_hbm.at[1, st], bufs[1].at[half], gsems[1]).wait()
    pltpu.make_async_copy(g1_hbm.at[st], bufs[2].at[half], gsems[2]).wait()
    pltpu.make_async_copy(dinv_hbm.at[st], dinv_v, gsems[3]).wait()
    pltpu.make_async_copy(w2_hbm, w2_v, isems[2]).wait()
    pltpu.make_async_copy(b1_hbm, b1_v, isems[3]).wait()

    def zrow(r, carry):
        s1 = bufs[0][r] + bufs[1][r] + bufs[2][r]
        bufs[3][r] = jnp.maximum(dinv_v[r] * s1 + b1_v[0], 0.0)
        return carry

    lax.fori_loop(0, STRIPE, zrow, 0, unroll=8)

    def mmrow(r, carry):
        zv = bufs[3][r]
        accs = [zv[j] * w2_v[j] for j in range(4)]
        for k in range(4, D_HID):
            accs[k % 4] = accs[k % 4] + zv[k] * w2_v[k]
        acc = (accs[0] + accs[1]) + (accs[2] + accs[3])
        bufs[0][r] = acc * dinv_v[r]
        return carry

    lax.fori_loop(0, STRIPE, mmrow, 0, unroll=4)
    pltpu.sync_copy(bufs[0].at[half], shared_g.at[st])
    pltpu.sync_copy(bufs[0].at[half], g2_hbm.at[st])
    pltpu.make_async_copy(src_hbm.at[w], src_v, isems[0]).wait()
    pltpu.make_async_copy(dst_hbm.at[w], dst_v, isems[1]).wait()
    plsc.subcore_barrier()

    _edge_phase(src_v, dst_v, bufs, gsems, ssems, shared, shared_g)
    plsc.subcore_barrier()
    _flush_shared(shared, sp_hbm, c, s)


def _tc_matmul1(x_p, W1):
    def body(x_ref, w_ref, h_ref):
        h_ref[...] = jnp.dot(x_ref[...], w_ref[...],
                             preferred_element_type=_f32)

    return pl.pallas_call(
        body,
        out_shape=jax.ShapeDtypeStruct((NPAD, D_HID), _f32),
    )(x_p, W1)


def _tc_out(sp2, g2, dinv, b2):
    rows = NPAD * D_HID // 128
    lane = jnp.arange(128, dtype=jnp.int32)
    grp = lane // D_HID
    sel = (lane[:, None] == grp[None, :] * D_HID).astype(_f32)
    blk = (grp[:, None] == grp[None, :]).astype(_f32)

    def body(sp_ref, g_ref, dinv_ref, b_ref, sel_ref, blk_ref, out_ref):
        o = dinv_ref[...] * (sp_ref[0] + sp_ref[1] + g_ref[...]) + b_ref[...]
        m = o
        for k in (1, 2, 4, 8):
            m = jnp.maximum(m, pltpu.roll(m, 128 - k, 1))
        mb = jax.lax.dot(m, sel_ref[...], preferred_element_type=_f32)
        e = jnp.exp(o - mb)
        ssum = jax.lax.dot(e, blk_ref[...], preferred_element_type=_f32)
        out_ref[...] = (o - mb) - jnp.log(ssum)

    res = pl.pallas_call(
        body,
        out_shape=jax.ShapeDtypeStruct((rows, 128), _f32),
    )(sp2.reshape(2, rows, 128), g2.reshape(rows, 128),
      dinv.reshape(rows, 128), jnp.tile(b2, (1, 128 // D_HID)), sel, blk)
    return res.reshape(NPAD, D_HID)


def kernel(x, edge_index, W1, b1, W2, b2):
    dstp = edge_index[1].astype(jnp.int32).reshape(NW, NSB, SBLK)

    degp = _deg_pass(dstp)
    srcp, dstp2, _ = lax.optimization_barrier(
        (edge_index[0].astype(jnp.int32).reshape(NW, NSB, SBLK), dstp, degp))

    x_p = jnp.pad(x, ((0, NPAD - N), (0, 0)))
    h1 = _tc_matmul1(x_p, W1)
    sp1, g1, dinv = _edge1(h1, degp, srcp, dstp2)
    sp2, g2 = _edge2(sp1, g1, dinv, W2, b1.reshape(1, D_HID),
                     srcp, dstp2)
    out = _tc_out(sp2, g2, dinv, b2.reshape(1, D_HID))
    return out[:N]


# speedup vs baseline: 1.2688x; 1.1830x over previous
import functools

import jax
import jax.numpy as jnp
from jax import lax
from jax.experimental import pallas as pl
from jax.experimental.pallas import tpu as pltpu
from jax.experimental.pallas import tpu_sc as plsc

N = 10000
E = 320000
D_FEAT = 128
D_HID = 16

NPAD = 10240
NW = 32
SBLK = 1000
NSB = 10
EPW = NSB * SBLK
STRIPE = NPAD // 16

_mesh = plsc.VectorSubcoreMesh(core_axis_name="c", subcore_axis_name="s")
_sc_params = pltpu.CompilerParams(use_tc_tiling_on_sc=False)
_f32 = jnp.float32


def _fill(buf, n, val):
    v = jnp.full((D_HID,), val, _f32)

    def row(r, carry):
        buf[r] = v
        return carry

    lax.fori_loop(0, n, row, 0, unroll=8)


def _flush_shared(shared, out_hbm, c, s):
    pltpu.sync_copy(shared.at[pl.ds(s * STRIPE, STRIPE)],
                    out_hbm.at[c, pl.ds(s * STRIPE, STRIPE)])


def _rsqrt_newton(d):
    i = lax.bitcast_convert_type(d, jnp.int32)
    y = lax.bitcast_convert_type(0x5F3759DF - (i >> 1), _f32)
    for _ in range(3):
        y = y * (1.5 - 0.5 * d * y * y)
    return y


def _edge_phase(src_v, dst_v, bufs, gsems, ssems, shared, shared_g):
    def gather(i):
        return pltpu.async_copy(shared_g.at[src_v.at[i]], bufs[i % 4],
                                gsems[i % 4])

    def scatter(i):
        return pltpu.async_copy(bufs[i % 4], shared.at[dst_v.at[i]],
                                ssems[i % 4], add=True)

    for i in range(3):
        gather(i)
    for i in range(NSB):
        if i + 3 < NSB:
            if i >= 1:
                pltpu.make_async_copy(bufs[(i - 1) % 4],
                                      shared.at[dst_v.at[i - 1]],
                                      ssems[(i - 1) % 4]).wait()
            gather(i + 3)
        pltpu.make_async_copy(shared_g.at[src_v.at[i]], bufs[i % 4],
                              gsems[i % 4]).wait()
        scatter(i)
    for i in range(NSB - 4, NSB):
        pltpu.make_async_copy(bufs[i % 4], shared.at[dst_v.at[i]],
                              ssems[i % 4]).wait()


@functools.partial(
    pl.kernel,
    out_type=jax.ShapeDtypeStruct((2, NPAD, D_HID), _f32),
    mesh=_mesh,
    scratch_types=[
        pltpu.VMEM((NSB, SBLK), jnp.int32),
        pltpu.VMEM((SBLK, D_HID), _f32),
        pltpu.VMEM((STRIPE, D_HID), _f32),
        pltpu.VMEM_SHARED((NPAD, D_HID), _f32),
        pltpu.SemaphoreType.DMA,
        pltpu.SemaphoreType.DMA,
    ],
    compiler_params=_sc_params,
)
def _deg_pass(dst_hbm, out_hbm, dst_v, ones_v, zbuf, shared, sem, isem):
    c = lax.axis_index("c")
    s = lax.axis_index("s")
    w = c * 16 + s
    for i in range(NSB):
        pltpu.async_copy(dst_hbm.at[pl.ds(w * EPW + i * SBLK, SBLK)],
                         dst_v.at[i], isem)
    _fill(ones_v, SBLK, 1.0)
    _fill(zbuf, STRIPE, 0.0)
    pltpu.sync_copy(zbuf, shared.at[pl.ds(s * STRIPE, STRIPE)])
    for i in range(NSB):
        pltpu.make_async_copy(dst_hbm.at[pl.ds(w * EPW + i * SBLK, SBLK)],
                              dst_v.at[i], isem).wait()
    plsc.subcore_barrier()

    for i in range(NSB):
        pltpu.async_copy(ones_v, shared.at[dst_v.at[i]], sem, add=True)
    for i in range(NSB):
        pltpu.make_async_copy(ones_v, shared.at[dst_v.at[i]], sem).wait()
    plsc.subcore_barrier()
    _flush_shared(shared, out_hbm, c, s)


@functools.partial(
    pl.kernel,
    out_type=(jax.ShapeDtypeStruct((2, NPAD, D_HID), _f32),
              jax.ShapeDtypeStruct((NPAD, D_HID), _f32),
              jax.ShapeDtypeStruct((NPAD, D_HID), _f32)),
    mesh=_mesh,
    scratch_types=[
        pltpu.VMEM((NSB, SBLK), jnp.int32),
        pltpu.VMEM((NSB, SBLK), jnp.int32),
        [pltpu.VMEM((SBLK, D_HID), _f32) for _ in range(4)],
        [pltpu.SemaphoreType.DMA for _ in range(4)],
        [pltpu.SemaphoreType.DMA for _ in range(4)],
        [pltpu.SemaphoreType.DMA for _ in range(2)],
        pltpu.VMEM_SHARED((NPAD, D_HID), _f32),
        pltpu.VMEM_SHARED((NPAD, D_HID), _f32),
    ],
    compiler_params=_sc_params,
)
def _edge1(h1_hbm, degp_hbm, src_hbm, dst_hbm,
           sp_hbm, g1_hbm, dinv_hbm,
           src_v, dst_v, bufs, gsems, ssems, isems, shared, shared_g):
    c = lax.axis_index("c")
    s = lax.axis_index("s")
    w = c * 16 + s
    st = pl.ds(s * STRIPE, STRIPE)
    half = pl.ds(0, STRIPE)
    for i in range(NSB):
        pltpu.async_copy(src_hbm.at[pl.ds(w * EPW + i * SBLK, SBLK)],
                         src_v.at[i], isems[0])
        pltpu.async_copy(dst_hbm.at[pl.ds(w * EPW + i * SBLK, SBLK)],
                         dst_v.at[i], isems[1])
    pltpu.async_copy(degp_hbm.at[0, st], bufs[0].at[half], gsems[0])
    pltpu.async_copy(degp_hbm.at[1, st], bufs[1].at[half], gsems[1])
    pltpu.async_copy(h1_hbm.at[st], bufs[2].at[half], gsems[2])
    _fill(bufs[3], STRIPE, 0.0)
    pltpu.sync_copy(bufs[3].at[half], shared.at[st])
    pltpu.make_async_copy(degp_hbm.at[0, st], bufs[0].at[half], gsems[0]).wait()
    pltpu.make_async_copy(degp_hbm.at[1, st], bufs[1].at[half], gsems[1]).wait()
    pltpu.make_async_copy(h1_hbm.at[st], bufs[2].at[half], gsems[2]).wait()

    def row(r, carry):
        d = bufs[0][r] + bufs[1][r] + 1.0
        dinv = _rsqrt_newton(d)
        bufs[0][r] = dinv
        bufs[3][r] = dinv * bufs[2][r]
        return carry

    lax.fori_loop(0, STRIPE, row, 0, unroll=8)
    pltpu.sync_copy(bufs[3].at[half], shared_g.at[st])
    pltpu.sync_copy(bufs[3].at[half], g1_hbm.at[st])
    pltpu.sync_copy(bufs[0].at[half], dinv_hbm.at[st])
    for i in range(NSB):
        pltpu.make_async_copy(src_hbm.at[pl.ds(w * EPW + i * SBLK, SBLK)],
                              src_v.at[i], isems[0]).wait()
        pltpu.make_async_copy(dst_hbm.at[pl.ds(w * EPW + i * SBLK, SBLK)],
                              dst_v.at[i], isems[1]).wait()
    plsc.subcore_barrier()

    _edge_phase(src_v, dst_v, bufs, gsems, ssems, shared, shared_g)
    plsc.subcore_barrier()
    _flush_shared(shared, sp_hbm, c, s)


@functools.partial(
    pl.kernel,
    out_type=(jax.ShapeDtypeStruct((2, NPAD, D_HID), _f32),
              jax.ShapeDtypeStruct((NPAD, D_HID), _f32)),
    mesh=_mesh,
    scratch_types=[
        pltpu.VMEM((NSB, SBLK), jnp.int32),
        pltpu.VMEM((NSB, SBLK), jnp.int32),
        [pltpu.VMEM((SBLK, D_HID), _f32) for _ in range(4)],
        pltpu.VMEM((STRIPE, D_HID), _f32),
        pltpu.VMEM((D_HID, D_HID), _f32),
        pltpu.VMEM((1, D_HID), _f32),
        [pltpu.SemaphoreType.DMA for _ in range(4)],
        [pltpu.SemaphoreType.DMA for _ in range(4)],
        [pltpu.SemaphoreType.DMA for _ in range(4)],
        pltpu.VMEM_SHARED((NPAD, D_HID), _f32),
        pltpu.VMEM_SHARED((NPAD, D_HID), _f32),
    ],
    compiler_params=_sc_params,
)
def _edge2(sp1_hbm, g1_hbm, dinv_hbm, w2_hbm, b1_hbm, src_hbm, dst_hbm,
           sp_hbm, g2_hbm,
           src_v, dst_v, bufs, dinv_v, w2_v, b1_v, gsems, ssems, isems,
           shared, shared_g):
    c = lax.axis_index("c")
    s = lax.axis_index("s")
    w = c * 16 + s
    st = pl.ds(s * STRIPE, STRIPE)
    half = pl.ds(0, STRIPE)
    for i in range(NSB):
        pltpu.async_copy(src_hbm.at[pl.ds(w * EPW + i * SBLK, SBLK)],
                         src_v.at[i], isems[0])
        pltpu.async_copy(dst_hbm.at[pl.ds(w * EPW + i * SBLK, SBLK)],
                         dst_v.at[i], isems[1])
    pltpu.async_copy(sp1_hbm.at[0, st], bufs[0].at[half], gsems[0])
    pltpu.async_copy(sp1_hbm.at[1, st], bufs[1].at[half], gsems[1])
    pltpu.async_copy(g1_hbm.at[st], bufs[2].at[half], gsems[2])
    pltpu.async_copy(dinv_hbm.at[st], dinv_v, gsems[3])
    pltpu.async_copy(w2_hbm, w2_v, isems[2])
    pltpu.async_copy(b1_hbm, b1_v, isems[3])
    _fill(bufs[3], STRIPE, 0.0)
    pltpu.sync_copy(bufs[3].at[half], shared.at[st])
    pltpu.make_async_copy(sp1_hbm.at[0, st], bufs[0].at[half], gsems[0]).wait()
    pltpu.make_async_copy(sp1_hbm.at[1, st], bufs[1].at[half], gsems[1]).wait()
    pltpu.make_async_copy(g1_hbm.at[st], bufs[2].at[half], gsems[2]).wait()
    pltpu.make_async_copy(dinv_hbm.at[st], dinv_v, gsems[3]).wait()
    pltpu.make_async_copy(w2_hbm, w2_v, isems[2]).wait()
    pltpu.make_async_copy(b1_hbm, b1_v, isems[3]).wait()

    def zrow(r, carry):
        s1 = bufs[0][r] + bufs[1][r] + bufs[2][r]
        bufs[3][r] = jnp.maximum(dinv_v[r] * s1 + b1_v[0], 0.0)
        return carry

    lax.fori_loop(0, STRIPE, zrow, 0, unroll=8)

    def mmrow(r, carry):
        zv = bufs[3][r]
        accs = [zv[j] * w2_v[j] for j in range(4)]
        for k in range(4, D_HID):
            accs[k % 4] = accs[k % 4] + zv[k] * w2_v[k]
        acc = (accs[0] + accs[1]) + (accs[2] + accs[3])
        bufs[0][r] = acc * dinv_v[r]
        return carry

    lax.fori_loop(0, STRIPE, mmrow, 0, unroll=4)
    pltpu.sync_copy(bufs[0].at[half], shared_g.at[st])
    pltpu.sync_copy(bufs[0].at[half], g2_hbm.at[st])
    for i in range(NSB):
        pltpu.make_async_copy(src_hbm.at[pl.ds(w * EPW + i * SBLK, SBLK)],
                              src_v.at[i], isems[0]).wait()
        pltpu.make_async_copy(dst_hbm.at[pl.ds(w * EPW + i * SBLK, SBLK)],
                              dst_v.at[i], isems[1]).wait()
    plsc.subcore_barrier()

    _edge_phase(src_v, dst_v, bufs, gsems, ssems, shared, shared_g)
    plsc.subcore_barrier()
    _flush_shared(shared, sp_hbm, c, s)


def _tc_detile(edge_index):
    rows = E // 128

    def body(e_ref, s_ref, d_ref):
        s_ref[...] = jnp.pad(e_ref[0].reshape(rows, 128), ((0, 4), (0, 0)))
        d_ref[...] = jnp.pad(e_ref[1].reshape(rows, 128), ((0, 4), (0, 0)))

    s_out, d_out = pl.pallas_call(
        body,
        out_shape=(jax.ShapeDtypeStruct((rows + 4, 128), jnp.int32),
                   jax.ShapeDtypeStruct((rows + 4, 128), jnp.int32)),
    )(edge_index)
    return s_out.reshape(-1), d_out.reshape(-1)


def _tc_matmul1(x_p, W1):
    def body(x_ref, w_ref, h_ref):
        h_ref[...] = jnp.dot(x_ref[...], w_ref[...],
                             preferred_element_type=_f32)

    return pl.pallas_call(
        body,
        out_shape=jax.ShapeDtypeStruct((NPAD, D_HID), _f32),
    )(x_p, W1)


def _tc_out(sp2, g2, dinv, b2):
    rows = NPAD * D_HID // 128
    lane = jnp.arange(128, dtype=jnp.int32)
    grp = lane // D_HID
    sel = (lane[:, None] == grp[None, :] * D_HID).astype(_f32)
    blk = (grp[:, None] == grp[None, :]).astype(_f32)

    def body(sp_ref, g_ref, dinv_ref, b_ref, sel_ref, blk_ref, out_ref):
        o = dinv_ref[...] * (sp_ref[0] + sp_ref[1] + g_ref[...]) + b_ref[...]
        m = o
        for k in (1, 2, 4, 8):
            m = jnp.maximum(m, pltpu.roll(m, 128 - k, 1))
        mb = jax.lax.dot(m, sel_ref[...], preferred_element_type=_f32)
        e = jnp.exp(o - mb)
        ssum = jax.lax.dot(e, blk_ref[...], preferred_element_type=_f32)
        out_ref[...] = (o - mb) - jnp.log(ssum)

    res = pl.pallas_call(
        body,
        out_shape=jax.ShapeDtypeStruct((rows, 128), _f32),
    )(sp2.reshape(2, rows, 128), g2.reshape(rows, 128),
      dinv.reshape(rows, 128), jnp.tile(b2, (1, 128 // D_HID)), sel, blk)
    return res.reshape(NPAD, D_HID)


def kernel(x, edge_index, W1, b1, W2, b2):
    src_f, dst_f = _tc_detile(edge_index.astype(jnp.int32))

    degp = _deg_pass(dst_f)
    x_p = jnp.pad(x, ((0, NPAD - N), (0, 0)))
    h1 = _tc_matmul1(x_p, W1)
    sp1, g1, dinv = _edge1(h1, degp, src_f, dst_f)
    sp2, g2 = _edge2(sp1, g1, dinv, W2, b1.reshape(1, D_HID),
                     src_f, dst_f)
    out = _tc_out(sp2, g2, dinv, b2.reshape(1, D_HID))
    return out[:N]


# speedup vs baseline: 1.2985x; 1.0234x over previous
import functools

import jax
import jax.numpy as jnp
from jax import lax
from jax.experimental import pallas as pl
from jax.experimental.pallas import tpu as pltpu
from jax.experimental.pallas import tpu_sc as plsc

N = 10000
E = 320000
D_FEAT = 128
D_HID = 16

NPAD = 10240
NW = 32
SBLK = 1000
NSB = 10
EPW = NSB * SBLK
STRIPE = NPAD // 16

_mesh = plsc.VectorSubcoreMesh(core_axis_name="c", subcore_axis_name="s")
_sc_params = pltpu.CompilerParams(use_tc_tiling_on_sc=False)
_f32 = jnp.float32


def _fill(buf, n, val):
    v = jnp.full((D_HID,), val, _f32)

    def row(r, carry):
        buf[r] = v
        return carry

    lax.fori_loop(0, n, row, 0, unroll=8)


def _flush_shared(shared, out_hbm, c, s):
    pltpu.sync_copy(shared.at[pl.ds(s * STRIPE, STRIPE)],
                    out_hbm.at[c, pl.ds(s * STRIPE, STRIPE)])


def _rsqrt_newton(d):
    i = lax.bitcast_convert_type(d, jnp.int32)
    y = lax.bitcast_convert_type(0x5F3759DF - (i >> 1), _f32)
    for _ in range(3):
        y = y * (1.5 - 0.5 * d * y * y)
    return y


def _edge_phase(src_v, dst_v, bufs, gsems, ssems, shared, shared_g):
    def gather(i):
        return pltpu.async_copy(shared_g.at[src_v.at[i]], bufs[i % 4],
                                gsems[i % 4])

    def scatter(i):
        return pltpu.async_copy(bufs[i % 4], shared.at[dst_v.at[i]],
                                ssems[i % 4], add=True)

    for i in range(3):
        gather(i)
    for i in range(NSB):
        if i + 3 < NSB:
            if i >= 1:
                pltpu.make_async_copy(bufs[(i - 1) % 4],
                                      shared.at[dst_v.at[i - 1]],
                                      ssems[(i - 1) % 4]).wait()
            gather(i + 3)
        pltpu.make_async_copy(shared_g.at[src_v.at[i]], bufs[i % 4],
                              gsems[i % 4]).wait()
        scatter(i)
    for i in range(NSB - 4, NSB):
        pltpu.make_async_copy(bufs[i % 4], shared.at[dst_v.at[i]],
                              ssems[i % 4]).wait()


@functools.partial(
    pl.kernel,
    out_type=jax.ShapeDtypeStruct((2, NPAD, D_HID), _f32),
    mesh=_mesh,
    scratch_types=[
        pltpu.VMEM((NSB, SBLK), jnp.int32),
        pltpu.VMEM((SBLK, D_HID), _f32),
        pltpu.VMEM((STRIPE, D_HID), _f32),
        pltpu.VMEM_SHARED((NPAD, D_HID), _f32),
        pltpu.SemaphoreType.DMA,
        pltpu.SemaphoreType.DMA,
    ],
    compiler_params=_sc_params,
)
def _deg_pass(dst_hbm, out_hbm, dst_v, ones_v, zbuf, shared, sem, isem):
    c = lax.axis_index("c")
    s = lax.axis_index("s")
    w = c * 16 + s
    for i in range(NSB):
        pltpu.async_copy(dst_hbm.at[pl.ds(w * EPW + i * SBLK, SBLK)],
                         dst_v.at[i], isem)
    _fill(ones_v, SBLK, 1.0)
    _fill(zbuf, STRIPE, 0.0)
    pltpu.sync_copy(zbuf, shared.at[pl.ds(s * STRIPE, STRIPE)])
    for i in range(NSB):
        pltpu.make_async_copy(dst_hbm.at[pl.ds(w * EPW + i * SBLK, SBLK)],
                              dst_v.at[i], isem).wait()
    plsc.subcore_barrier()

    for i in range(NSB):
        pltpu.async_copy(ones_v, shared.at[dst_v.at[i]], sem, add=True)
    for i in range(NSB):
        pltpu.make_async_copy(ones_v, shared.at[dst_v.at[i]], sem).wait()
    plsc.subcore_barrier()
    _flush_shared(shared, out_hbm, c, s)


@functools.partial(
    pl.kernel,
    out_type=(jax.ShapeDtypeStruct((2, NPAD, D_HID), _f32),
              jax.ShapeDtypeStruct((NPAD, D_HID), _f32),
              jax.ShapeDtypeStruct((NPAD, D_HID), _f32)),
    mesh=_mesh,
    scratch_types=[
        pltpu.VMEM((NSB, SBLK), jnp.int32),
        pltpu.VMEM((NSB, SBLK), jnp.int32),
        [pltpu.VMEM((SBLK, D_HID), _f32) for _ in range(4)],
        [pltpu.SemaphoreType.DMA for _ in range(4)],
        [pltpu.SemaphoreType.DMA for _ in range(4)],
        [pltpu.SemaphoreType.DMA for _ in range(2)],
        pltpu.VMEM_SHARED((NPAD, D_HID), _f32),
        pltpu.VMEM_SHARED((NPAD, D_HID), _f32),
    ],
    compiler_params=_sc_params,
)
def _edge1(h1_hbm, degp_hbm, src_hbm, dst_hbm,
           sp_hbm, g1_hbm, dinv_hbm,
           src_v, dst_v, bufs, gsems, ssems, isems, shared, shared_g):
    c = lax.axis_index("c")
    s = lax.axis_index("s")
    w = c * 16 + s
    st = pl.ds(s * STRIPE, STRIPE)
    half = pl.ds(0, STRIPE)
    for i in range(NSB):
        pltpu.async_copy(src_hbm.at[pl.ds(w * EPW + i * SBLK, SBLK)],
                         src_v.at[i], isems[0])
        pltpu.async_copy(dst_hbm.at[pl.ds(w * EPW + i * SBLK, SBLK)],
                         dst_v.at[i], isems[1])
    pltpu.async_copy(degp_hbm.at[0, st], bufs[0].at[half], gsems[0])
    pltpu.async_copy(degp_hbm.at[1, st], bufs[1].at[half], gsems[1])
    pltpu.async_copy(h1_hbm.at[st], bufs[2].at[half], gsems[2])
    _fill(bufs[3], STRIPE, 0.0)
    pltpu.sync_copy(bufs[3].at[half], shared.at[st])
    pltpu.make_async_copy(degp_hbm.at[0, st], bufs[0].at[half], gsems[0]).wait()
    pltpu.make_async_copy(degp_hbm.at[1, st], bufs[1].at[half], gsems[1]).wait()
    pltpu.make_async_copy(h1_hbm.at[st], bufs[2].at[half], gsems[2]).wait()

    def row(r, carry):
        d = bufs[0][r] + bufs[1][r] + 1.0
        dinv = _rsqrt_newton(d)
        bufs[0][r] = dinv
        bufs[3][r] = dinv * bufs[2][r]
        return carry

    lax.fori_loop(0, STRIPE, row, 0, unroll=8)
    pltpu.sync_copy(bufs[3].at[half], shared_g.at[st])
    pltpu.sync_copy(bufs[3].at[half], g1_hbm.at[st])
    pltpu.sync_copy(bufs[0].at[half], dinv_hbm.at[st])
    for i in range(NSB):
        pltpu.make_async_copy(src_hbm.at[pl.ds(w * EPW + i * SBLK, SBLK)],
                              src_v.at[i], isems[0]).wait()
        pltpu.make_async_copy(dst_hbm.at[pl.ds(w * EPW + i * SBLK, SBLK)],
                              dst_v.at[i], isems[1]).wait()
    plsc.subcore_barrier()

    _edge_phase(src_v, dst_v, bufs, gsems, ssems, shared, shared_g)
    plsc.subcore_barrier()
    _flush_shared(shared, sp_hbm, c, s)


@functools.partial(
    pl.kernel,
    out_type=(jax.ShapeDtypeStruct((2, NPAD, D_HID), _f32),
              jax.ShapeDtypeStruct((NPAD, D_HID), _f32)),
    mesh=_mesh,
    scratch_types=[
        pltpu.VMEM((NSB, SBLK), jnp.int32),
        pltpu.VMEM((NSB, SBLK), jnp.int32),
        [pltpu.VMEM((SBLK, D_HID), _f32) for _ in range(4)],
        pltpu.VMEM((STRIPE, D_HID), _f32),
        pltpu.VMEM((D_HID, D_HID), _f32),
        pltpu.VMEM((1, D_HID), _f32),
        [pltpu.SemaphoreType.DMA for _ in range(4)],
        [pltpu.SemaphoreType.DMA for _ in range(4)],
        [pltpu.SemaphoreType.DMA for _ in range(4)],
        pltpu.VMEM_SHARED((NPAD, D_HID), _f32),
        pltpu.VMEM_SHARED((NPAD, D_HID), _f32),
    ],
    compiler_params=_sc_params,
)
def _edge2(sp1_hbm, g1_hbm, dinv_hbm, w2_hbm, b1_hbm, src_hbm, dst_hbm,
           sp_hbm, g2_hbm,
           src_v, dst_v, bufs, dinv_v, w2_v, b1_v, gsems, ssems, isems,
           shared, shared_g):
    c = lax.axis_index("c")
    s = lax.axis_index("s")
    w = c * 16 + s
    st = pl.ds(s * STRIPE, STRIPE)
    half = pl.ds(0, STRIPE)
    for i in range(NSB):
        pltpu.async_copy(src_hbm.at[pl.ds(w * EPW + i * SBLK, SBLK)],
                         src_v.at[i], isems[0])
        pltpu.async_copy(dst_hbm.at[pl.ds(w * EPW + i * SBLK, SBLK)],
                         dst_v.at[i], isems[1])
    pltpu.async_copy(sp1_hbm.at[0, st], bufs[0].at[half], gsems[0])
    pltpu.async_copy(sp1_hbm.at[1, st], bufs[1].at[half], gsems[1])
    pltpu.async_copy(g1_hbm.at[st], bufs[2].at[half], gsems[2])
    pltpu.async_copy(dinv_hbm.at[st], dinv_v, gsems[3])
    pltpu.async_copy(w2_hbm, w2_v, isems[2])
    pltpu.async_copy(b1_hbm, b1_v, isems[3])
    _fill(bufs[3], STRIPE, 0.0)
    pltpu.sync_copy(bufs[3].at[half], shared.at[st])
    pltpu.make_async_copy(sp1_hbm.at[0, st], bufs[0].at[half], gsems[0]).wait()
    pltpu.make_async_copy(sp1_hbm.at[1, st], bufs[1].at[half], gsems[1]).wait()
    pltpu.make_async_copy(g1_hbm.at[st], bufs[2].at[half], gsems[2]).wait()
    pltpu.make_async_copy(dinv_hbm.at[st], dinv_v, gsems[3]).wait()
    pltpu.make_async_copy(w2_hbm, w2_v, isems[2]).wait()
    pltpu.make_async_copy(b1_hbm, b1_v, isems[3]).wait()

    def zrow(r, carry):
        s1 = bufs[0][r] + bufs[1][r] + bufs[2][r]
        bufs[3][r] = jnp.maximum(dinv_v[r] * s1 + b1_v[0], 0.0)
        return carry

    lax.fori_loop(0, STRIPE, zrow, 0, unroll=8)

    def mmrow(r, carry):
        zv = bufs[3][r]
        accs = [zv[j] * w2_v[j] for j in range(4)]
        for k in range(4, D_HID):
            accs[k % 4] = accs[k % 4] + zv[k] * w2_v[k]
        acc = (accs[0] + accs[1]) + (accs[2] + accs[3])
        bufs[0][r] = acc * dinv_v[r]
        return carry

    lax.fori_loop(0, STRIPE, mmrow, 0, unroll=4)
    pltpu.sync_copy(bufs[0].at[half], shared_g.at[st])
    pltpu.sync_copy(bufs[0].at[half], g2_hbm.at[st])
    for i in range(NSB):
        pltpu.make_async_copy(src_hbm.at[pl.ds(w * EPW + i * SBLK, SBLK)],
                              src_v.at[i], isems[0]).wait()
        pltpu.make_async_copy(dst_hbm.at[pl.ds(w * EPW + i * SBLK, SBLK)],
                              dst_v.at[i], isems[1]).wait()
    plsc.subcore_barrier()

    _edge_phase(src_v, dst_v, bufs, gsems, ssems, shared, shared_g)
    plsc.subcore_barrier()
    _flush_shared(shared, sp_hbm, c, s)


def _tc_detile(edge_index):
    rows = E // 128

    def body(e_ref, s_ref, d_ref):
        s_ref[...] = jnp.pad(e_ref[0].reshape(rows, 128), ((0, 4), (0, 0)))
        d_ref[...] = jnp.pad(e_ref[1].reshape(rows, 128), ((0, 4), (0, 0)))

    s_out, d_out = pl.pallas_call(
        body,
        out_shape=(jax.ShapeDtypeStruct((rows + 4, 128), jnp.int32),
                   jax.ShapeDtypeStruct((rows + 4, 128), jnp.int32)),
    )(edge_index)
    return s_out.reshape(-1), d_out.reshape(-1)


def _tc_matmul1(x, W1):
    def body(x_ref, w_ref, h_ref):
        h_ref[...] = jnp.pad(
            jnp.dot(x_ref[...], w_ref[...], preferred_element_type=_f32),
            ((0, NPAD - N), (0, 0)))

    return pl.pallas_call(
        body,
        out_shape=jax.ShapeDtypeStruct((NPAD, D_HID), _f32),
    )(x, W1)


def _tc_out(sp2, g2, dinv, b2):
    rows = NPAD * D_HID // 128
    lane = jnp.arange(128, dtype=jnp.int32)
    grp = lane // D_HID
    sel = (lane[:, None] == grp[None, :] * D_HID).astype(_f32)
    blk = (grp[:, None] == grp[None, :]).astype(_f32)

    def body(sp_ref, g_ref, dinv_ref, b_ref, sel_ref, blk_ref, out_ref):
        o = dinv_ref[...] * (sp_ref[0] + sp_ref[1] + g_ref[...]) + b_ref[...]
        m = o
        for k in (1, 2, 4, 8):
            m = jnp.maximum(m, pltpu.roll(m, 128 - k, 1))
        mb = jax.lax.dot(m, sel_ref[...], preferred_element_type=_f32)
        e = jnp.exp(o - mb)
        ssum = jax.lax.dot(e, blk_ref[...], preferred_element_type=_f32)
        out_ref[...] = ((o - mb) - jnp.log(ssum))[:N * D_HID // 128]

    res = pl.pallas_call(
        body,
        out_shape=jax.ShapeDtypeStruct((N * D_HID // 128, 128), _f32),
    )(sp2.reshape(2, rows, 128), g2.reshape(rows, 128),
      dinv.reshape(rows, 128), jnp.tile(b2, (1, 128 // D_HID)), sel, blk)
    return res.reshape(N, D_HID)


def kernel(x, edge_index, W1, b1, W2, b2):
    src_f, dst_f = _tc_detile(edge_index.astype(jnp.int32))

    degp = _deg_pass(dst_f)
    h1 = _tc_matmul1(x, W1)
    sp1, g1, dinv = _edge1(h1, degp, src_f, dst_f)
    sp2, g2 = _edge2(sp1, g1, dinv, W2, b1.reshape(1, D_HID),
                     src_f, dst_f)
    return _tc_out(sp2, g2, dinv, b2.reshape(1, D_HID))
